# baseline (speedup 1.0000x reference)
import functools

import jax
import jax.numpy as jnp
from jax import lax
from jax.experimental import pallas as pl
from jax.experimental.pallas import tpu as pltpu
from jax.experimental.pallas import tpu_sc as plsc

N = 10000
D = 128
E = 320000

NC = 2
NS = 16
NW = NC * NS
NP = 10240
RPT = NP // NS
EPW = 10752
EP = NW * EPW
PAD_IDX = N

CHD = 128
NCHD = EPW // CHD
CH = 128
NCH = EPW // CH
NBUF = 3
IB = 4
NACC = 10104
RPTA = 632
RPTL = NACC - 15 * RPTA


def _deg_body(dst_hbm, ones_hbm, zeros_hbm, deg_out, dst_all, ones_v, acc):
    c = lax.axis_index("c")
    s = lax.axis_index("s")
    wid = s * NC + c
    pltpu.sync_copy(zeros_hbm, acc.at[pl.ds(s * RPT, RPT)])
    pltpu.sync_copy(ones_hbm, ones_v)
    pltpu.sync_copy(dst_hbm.at[wid], dst_all)
    plsc.subcore_barrier()

    def body(i, carry):
        pltpu.sync_copy(ones_v, acc.at[dst_all.at[i]], add=True)
        return carry

    lax.fori_loop(0, NCHD, body, 0)
    plsc.subcore_barrier()
    pltpu.sync_copy(acc.at[pl.ds(s * RPT, RPT)],
                    deg_out.at[c, pl.ds(s * RPT, RPT)])


@functools.cache
def _deg_kernel():
    mesh = plsc.VectorSubcoreMesh(core_axis_name="c", subcore_axis_name="s",
                                  num_cores=NC, num_subcores=NS)
    return pl.kernel(
        _deg_body,
        out_type=jax.ShapeDtypeStruct((NC, NP, D), jnp.float32),
        mesh=mesh,
        scratch_types=[
            pltpu.VMEM((NCHD, CHD), jnp.int32),
            pltpu.VMEM((CHD, D), jnp.float32),
            pltpu.VMEM_SHARED((NP, D), jnp.float32),
        ],
    )


def _spmm_body(y_hbm, src_hbm, dst_hbm, zeros_hbm, s_out,
               src_v, dst_v, rows, isem_s, isem_d, gsem, ssem, acc):
    c = lax.axis_index("c")
    s = lax.axis_index("s")
    wid = s * NC + c
    @pl.when(s < 15)
    def _():
        pltpu.sync_copy(zeros_hbm, acc.at[pl.ds(s * RPTA, RPTA)])

    @pl.when(s == 15)
    def _():
        pltpu.sync_copy(zeros_hbm.at[pl.ds(0, RPTL)],
                        acc.at[pl.ds(15 * RPTA, RPTL)])

    plsc.subcore_barrier()

    def idx_start(i, b):
        pltpu.async_copy(src_hbm.at[c, wid, i], src_v[b], isem_s[b])
        pltpu.async_copy(dst_hbm.at[wid, i], dst_v[b], isem_d[b])

    def idx_wait(i, b):
        pltpu.make_async_copy(src_hbm.at[c, wid, i], src_v[b], isem_s[b]).wait()
        pltpu.make_async_copy(dst_hbm.at[wid, i], dst_v[b], isem_d[b]).wait()

    def gather_start(b3, b4):
        pltpu.async_copy(y_hbm.at[src_v[b4]], rows[b3], gsem[b3])

    def gather_wait(b3, b4):
        pltpu.make_async_copy(y_hbm.at[src_v[b4]], rows[b3], gsem[b3]).wait()

    def scatter_start(b3, b4):
        pltpu.async_copy(rows[b3], acc.at[dst_v[b4]], ssem[b3], add=True)

    def scatter_wait(b3, b4):
        pltpu.make_async_copy(rows[b3], acc.at[dst_v[b4]], ssem[b3]).wait()

    idx_start(0, 0)
    idx_start(1, 1)
    idx_wait(0, 0)
    gather_start(0, 0)
    idx_start(2, 2)
    idx_wait(1, 1)
    gather_start(1, 1)

    def body(j, carry):
        for k in range(12):
            i = j * 12 + k
            b3 = k % 3
            b4 = k % 4
            gather_wait(b3, b4)
            scatter_start(b3, b4)

            @pl.when(i + 2 < NCH)
            def _():
                idx_wait(i + 2, (k + 2) % 4)

            @pl.when(i >= 1)
            def _():
                scatter_wait((k + 2) % 3, (k + 3) % 4)

            @pl.when(i + 2 < NCH)
            def _():
                gather_start((k + 2) % 3, (k + 2) % 4)

            @pl.when(i + 3 < NCH)
            def _():
                idx_start(i + 3, (k + 3) % 4)

        return carry

    lax.fori_loop(0, NCH // 12, body, 0)
    scatter_wait((NCH - 1) % 3, (NCH - 1) % 4)
    plsc.subcore_barrier()

    @pl.when(s < 15)
    def _():
        pltpu.sync_copy(acc.at[pl.ds(s * RPTA, RPTA)],
                        s_out.at[c, pl.ds(s * RPTA, RPTA)])

    @pl.when(s == 15)
    def _():
        pltpu.sync_copy(acc.at[pl.ds(15 * RPTA, RPTL)],
                        s_out.at[c, pl.ds(15 * RPTA, RPTL)])


@functools.cache
def _spmm_kernel():
    mesh = plsc.VectorSubcoreMesh(core_axis_name="c", subcore_axis_name="s",
                                  num_cores=NC, num_subcores=NS)
    return pl.kernel(
        _spmm_body,
        out_type=jax.ShapeDtypeStruct((NC, NP, D), jnp.float32),
        mesh=mesh,
        scratch_types=[
            [pltpu.VMEM((CH,), jnp.int32) for _ in range(IB)],
            [pltpu.VMEM((CH,), jnp.int32) for _ in range(IB)],
            [pltpu.VMEM((CH, D), jnp.float32) for _ in range(NBUF)],
            [pltpu.SemaphoreType.DMA for _ in range(IB)],
            [pltpu.SemaphoreType.DMA for _ in range(IB)],
            [pltpu.SemaphoreType.DMA for _ in range(NBUF)],
            [pltpu.SemaphoreType.DMA for _ in range(NBUF)],
            pltpu.VMEM_SHARED((NACC, D), jnp.float32),
        ],
    )


BR = 512
GRID = NP // BR


def _dis(deg_blk):
    return lax.rsqrt(deg_blk[0, :, 0:1] + deg_blk[1, :, 0:1] + 1.0)


def _mm1_body(deg_ref, x_ref, w_ref, y_ref, dis_ref):
    dis = _dis(deg_ref[...])
    res = dis * jnp.dot(x_ref[...], w_ref[...],
                        preferred_element_type=jnp.float32)
    y_ref[0] = res
    y_ref[1] = res
    dis_ref[...] = jnp.broadcast_to(dis, (BR, D))


_mm1 = pl.pallas_call(
    _mm1_body,
    grid=(GRID,),
    in_specs=[
        pl.BlockSpec((2, BR, D), lambda i: (0, i, 0)),
        pl.BlockSpec((BR, D), lambda i: (i, 0)),
        pl.BlockSpec((D, D), lambda i: (0, 0)),
    ],
    out_specs=[pl.BlockSpec((2, BR, D), lambda i: (0, i, 0)),
               pl.BlockSpec((BR, D), lambda i: (i, 0))],
    out_shape=[jax.ShapeDtypeStruct((NC, NP, D), jnp.float32),
               jax.ShapeDtypeStruct((NP, D), jnp.float32)],
)


def _mid_body(dis_ref, s_ref, y_ref, b_ref, w_ref, o_ref):
    dis = dis_ref[...]
    h = dis * (s_ref[0] + s_ref[1] + y_ref[0]) + b_ref[...]
    h = jnp.maximum(h, 0.0)
    res = dis * jnp.dot(h, w_ref[...], preferred_element_type=jnp.float32)
    o_ref[0] = res
    o_ref[1] = res


_mid = pl.pallas_call(
    _mid_body,
    grid=(GRID,),
    in_specs=[
        pl.BlockSpec((BR, D), lambda i: (i, 0)),
        pl.BlockSpec((2, BR, D), lambda i: (0, i, 0)),
        pl.BlockSpec((1, BR, D), lambda i: (0, i, 0)),
        pl.BlockSpec((1, D), lambda i: (0, 0)),
        pl.BlockSpec((D, D), lambda i: (0, 0)),
    ],
    out_specs=pl.BlockSpec((2, BR, D), lambda i: (0, i, 0)),
    out_shape=jax.ShapeDtypeStruct((NC, NP, D), jnp.float32),
)


def _fin_body(dis_ref, s_ref, y_ref, b_ref, o_ref):
    o_ref[...] = dis_ref[...] * (s_ref[0] + s_ref[1] + y_ref[0]) + b_ref[...]


_fin = pl.pallas_call(
    _fin_body,
    grid=(GRID,),
    in_specs=[
        pl.BlockSpec((BR, D), lambda i: (i, 0)),
        pl.BlockSpec((2, BR, D), lambda i: (0, i, 0)),
        pl.BlockSpec((1, BR, D), lambda i: (0, i, 0)),
        pl.BlockSpec((1, D), lambda i: (0, 0)),
    ],
    out_specs=pl.BlockSpec((BR, D), lambda i: (i, 0)),
    out_shape=jax.ShapeDtypeStruct((NP, D), jnp.float32),
)


def kernel(x, edge_index, W1, b1, W2, b2, W3, b3):
    src = edge_index[0]
    dst = edge_index[1]
    pad_e = PAD_IDX + jnp.arange(EP - E, dtype=jnp.int32) % (NACC - N)
    src_f = jnp.concatenate([src, pad_e]).reshape(NW, NCH, CH)
    src_p = jnp.stack([src_f, src_f + NP])
    dst_p = jnp.concatenate([dst, pad_e]).reshape(NW, NCH, CH)
    dst_w = jnp.concatenate([dst, pad_e]).reshape(NW, NCHD, CHD)
    x_p = jnp.zeros((NP, D), jnp.float32).at[:N].set(x)
    onesD = jnp.ones((CHD, D), jnp.float32)
    zerosD = jnp.zeros((RPT, D), jnp.float32)
    zerosA = jnp.zeros((RPTA, D), jnp.float32)

    deg = _deg_kernel()(dst_w, onesD, zerosD)
    spmm = _spmm_kernel()
    y1, dis = _mm1(deg, x_p, W1)
    s1 = spmm(y1.reshape(NC * NP, D), src_p, dst_p, zerosA)
    y2 = _mid(dis, s1, y1, b1.reshape(1, D), W2)
    s2 = spmm(y2.reshape(NC * NP, D), src_p, dst_p, zerosA)
    y3 = _mid(dis, s2, y2, b2.reshape(1, D), W3)
    s3 = spmm(y3.reshape(NC * NP, D), src_p, dst_p, zerosA)
    out = _fin(dis, s3, y3, b3.reshape(1, D))
    return out[:N]


# speedup vs baseline: 1.0466x; 1.0466x over previous
import functools

import jax
import jax.numpy as jnp
from jax import lax
from jax.experimental import pallas as pl
from jax.experimental.pallas import tpu as pltpu
from jax.experimental.pallas import tpu_sc as plsc

N = 10000
D = 128
E = 320000

NC = 2
NS = 16
NW = NC * NS
NP = 10240
RPT = NP // NS
EPW = 10752
EP = NW * EPW
PAD_IDX = N

CHD = 128
NCHD = EPW // CHD
CH = 128
NCH = EPW // CH
NBUF = 3
IB = 4
NACC = 10104
RPTA = 632
RPTL = NACC - 15 * RPTA


def _deg_body(dst_hbm, ones_hbm, zeros_hbm, deg_out, dst_all, ones_v, acc):
    c = lax.axis_index("c")
    s = lax.axis_index("s")
    wid = s * NC + c
    pltpu.sync_copy(zeros_hbm, acc.at[pl.ds(s * RPT, RPT)])
    pltpu.sync_copy(ones_hbm, ones_v)
    pltpu.sync_copy(dst_hbm.at[wid], dst_all)
    plsc.subcore_barrier()

    def body(i, carry):
        pltpu.sync_copy(ones_v, acc.at[dst_all.at[i]], add=True)
        return carry

    lax.fori_loop(0, NCHD, body, 0)
    plsc.subcore_barrier()
    pltpu.sync_copy(acc.at[pl.ds(s * RPT, RPT)],
                    deg_out.at[c, pl.ds(s * RPT, RPT)])


@functools.cache
def _deg_kernel():
    mesh = plsc.VectorSubcoreMesh(core_axis_name="c", subcore_axis_name="s",
                                  num_cores=NC, num_subcores=NS)
    return pl.kernel(
        _deg_body,
        out_type=jax.ShapeDtypeStruct((NC, NP, D), jnp.float32),
        mesh=mesh,
        scratch_types=[
            pltpu.VMEM((NCHD, CHD), jnp.int32),
            pltpu.VMEM((CHD, D), jnp.float32),
            pltpu.VMEM_SHARED((NP, D), jnp.float32),
        ],
    )


def _spmm_body(y_hbm, src_hbm, dst_hbm, zeros_hbm, s_out,
               src_v, dst_v, rows, isem_s, isem_d, gsem, ssem, acc):
    c = lax.axis_index("c")
    s = lax.axis_index("s")
    wid = s * NC + c
    @pl.when(s < 15)
    def _():
        pltpu.sync_copy(zeros_hbm, acc.at[pl.ds(s * RPTA, RPTA)])

    @pl.when(s == 15)
    def _():
        pltpu.sync_copy(zeros_hbm.at[pl.ds(0, RPTL)],
                        acc.at[pl.ds(15 * RPTA, RPTL)])

    plsc.subcore_barrier()

    def idx_start(i, b):
        pltpu.async_copy(src_hbm.at[c, wid, i], src_v[b], isem_s[b])
        pltpu.async_copy(dst_hbm.at[wid, i], dst_v[b], isem_d[b])

    def idx_wait(i, b):
        pltpu.make_async_copy(src_hbm.at[c, wid, i], src_v[b], isem_s[b]).wait()
        pltpu.make_async_copy(dst_hbm.at[wid, i], dst_v[b], isem_d[b]).wait()

    def gather_start(b3, b4):
        pltpu.async_copy(y_hbm.at[src_v[b4]], rows[b3], gsem[b3])

    def gather_wait(b3, b4):
        pltpu.make_async_copy(y_hbm.at[src_v[b4]], rows[b3], gsem[b3]).wait()

    def scatter_start(b3, b4):
        pltpu.async_copy(rows[b3], acc.at[dst_v[b4]], ssem[b3], add=True)

    def scatter_wait(b3, b4):
        pltpu.make_async_copy(rows[b3], acc.at[dst_v[b4]], ssem[b3]).wait()

    idx_start(0, 0)
    idx_start(1, 1)
    idx_wait(0, 0)
    gather_start(0, 0)
    idx_start(2, 2)
    idx_wait(1, 1)
    gather_start(1, 1)

    def body(j, carry):
        for k in range(12):
            i = j * 12 + k
            b3 = k % 3
            b4 = k % 4
            gather_wait(b3, b4)
            scatter_start(b3, b4)

            @pl.when(i + 2 < NCH)
            def _():
                idx_wait(i + 2, (k + 2) % 4)

            @pl.when(i >= 1)
            def _():
                scatter_wait((k + 2) % 3, (k + 3) % 4)

            @pl.when(i + 2 < NCH)
            def _():
                gather_start((k + 2) % 3, (k + 2) % 4)

            @pl.when(i + 3 < NCH)
            def _():
                idx_start(i + 3, (k + 3) % 4)

        return carry

    lax.fori_loop(0, NCH // 12, body, 0)
    scatter_wait((NCH - 1) % 3, (NCH - 1) % 4)
    plsc.subcore_barrier()

    @pl.when(s < 15)
    def _():
        pltpu.sync_copy(acc.at[pl.ds(s * RPTA, RPTA)],
                        s_out.at[c, pl.ds(s * RPTA, RPTA)])

    @pl.when(s == 15)
    def _():
        pltpu.sync_copy(acc.at[pl.ds(15 * RPTA, RPTL)],
                        s_out.at[c, pl.ds(15 * RPTA, RPTL)])


@functools.cache
def _spmm_kernel():
    mesh = plsc.VectorSubcoreMesh(core_axis_name="c", subcore_axis_name="s",
                                  num_cores=NC, num_subcores=NS)
    return pl.kernel(
        _spmm_body,
        out_type=jax.ShapeDtypeStruct((NC, NP, D), jnp.float32),
        mesh=mesh,
        scratch_types=[
            [pltpu.VMEM((CH,), jnp.int32) for _ in range(IB)],
            [pltpu.VMEM((CH,), jnp.int32) for _ in range(IB)],
            [pltpu.VMEM((CH, D), jnp.float32) for _ in range(NBUF)],
            [pltpu.SemaphoreType.DMA for _ in range(IB)],
            [pltpu.SemaphoreType.DMA for _ in range(IB)],
            [pltpu.SemaphoreType.DMA for _ in range(NBUF)],
            [pltpu.SemaphoreType.DMA for _ in range(NBUF)],
            pltpu.VMEM_SHARED((NACC, D), jnp.float32),
        ],
    )


BR = 1024
GRID = NP // BR


def _dis(deg_blk):
    return lax.rsqrt(deg_blk[0, :, 0:1] + deg_blk[1, :, 0:1] + 1.0)


def _mm1_body(deg_ref, x_ref, w_ref, y_ref, dis_ref):
    dis = _dis(deg_ref[...])
    res = dis * jnp.dot(x_ref[...], w_ref[...],
                        preferred_element_type=jnp.float32)
    y_ref[0] = res
    y_ref[1] = res
    dis_ref[...] = jnp.broadcast_to(dis, (BR, D))


_mm1 = pl.pallas_call(
    _mm1_body,
    grid=(GRID,),
    in_specs=[
        pl.BlockSpec((2, BR, D), lambda i: (0, i, 0)),
        pl.BlockSpec((BR, D), lambda i: (i, 0)),
        pl.BlockSpec((D, D), lambda i: (0, 0)),
    ],
    out_specs=[pl.BlockSpec((2, BR, D), lambda i: (0, i, 0)),
               pl.BlockSpec((BR, D), lambda i: (i, 0))],
    out_shape=[jax.ShapeDtypeStruct((NC, NP, D), jnp.float32),
               jax.ShapeDtypeStruct((NP, D), jnp.float32)],
)


def _mid_body(dis_ref, s_ref, y_ref, b_ref, w_ref, o_ref):
    dis = dis_ref[...]
    h = dis * (s_ref[0] + s_ref[1] + y_ref[0]) + b_ref[...]
    h = jnp.maximum(h, 0.0)
    res = dis * jnp.dot(h, w_ref[...], preferred_element_type=jnp.float32)
    o_ref[0] = res
    o_ref[1] = res


_mid = pl.pallas_call(
    _mid_body,
    grid=(GRID,),
    in_specs=[
        pl.BlockSpec((BR, D), lambda i: (i, 0)),
        pl.BlockSpec((2, BR, D), lambda i: (0, i, 0)),
        pl.BlockSpec((1, BR, D), lambda i: (0, i, 0)),
        pl.BlockSpec((1, D), lambda i: (0, 0)),
        pl.BlockSpec((D, D), lambda i: (0, 0)),
    ],
    out_specs=pl.BlockSpec((2, BR, D), lambda i: (0, i, 0)),
    out_shape=jax.ShapeDtypeStruct((NC, NP, D), jnp.float32),
)


def _fin_body(dis_ref, s_ref, y_ref, b_ref, o_ref):
    o_ref[...] = dis_ref[...] * (s_ref[0] + s_ref[1] + y_ref[0]) + b_ref[...]


_fin = pl.pallas_call(
    _fin_body,
    grid=(GRID,),
    in_specs=[
        pl.BlockSpec((BR, D), lambda i: (i, 0)),
        pl.BlockSpec((2, BR, D), lambda i: (0, i, 0)),
        pl.BlockSpec((1, BR, D), lambda i: (0, i, 0)),
        pl.BlockSpec((1, D), lambda i: (0, 0)),
    ],
    out_specs=pl.BlockSpec((BR, D), lambda i: (i, 0)),
    out_shape=jax.ShapeDtypeStruct((NP, D), jnp.float32),
)


def kernel(x, edge_index, W1, b1, W2, b2, W3, b3):
    src = edge_index[0]
    dst = edge_index[1]
    pad_e = PAD_IDX + jnp.arange(EP - E, dtype=jnp.int32) % (NACC - N)
    src_f = jnp.concatenate([src, pad_e]).reshape(NW, NCH, CH)
    src_p = jnp.stack([src_f, src_f + NP])
    dst_p = jnp.concatenate([dst, pad_e]).reshape(NW, NCH, CH)
    dst_w = jnp.concatenate([dst, pad_e]).reshape(NW, NCHD, CHD)
    x_p = jnp.zeros((NP, D), jnp.float32).at[:N].set(x)
    onesD = jnp.ones((CHD, D), jnp.float32)
    zerosD = jnp.zeros((RPT, D), jnp.float32)
    zerosA = jnp.zeros((RPTA, D), jnp.float32)

    deg = _deg_kernel()(dst_w, onesD, zerosD)
    spmm = _spmm_kernel()
    y1, dis = _mm1(deg, x_p, W1)
    s1 = spmm(y1.reshape(NC * NP, D), src_p, dst_p, zerosA)
    y2 = _mid(dis, s1, y1, b1.reshape(1, D), W2)
    s2 = spmm(y2.reshape(NC * NP, D), src_p, dst_p, zerosA)
    y3 = _mid(dis, s2, y2, b2.reshape(1, D), W3)
    s3 = spmm(y3.reshape(NC * NP, D), src_p, dst_p, zerosA)
    out = _fin(dis, s3, y3, b3.reshape(1, D))
    return out[:N]


# speedup vs baseline: 1.0609x; 1.0137x over previous
import functools

import jax
import jax.numpy as jnp
from jax import lax
from jax.experimental import pallas as pl
from jax.experimental.pallas import tpu as pltpu
from jax.experimental.pallas import tpu_sc as plsc

N = 10000
D = 128
E = 320000

NC = 2
NS = 16
NW = NC * NS
NP = 10240
RPT = NP // NS
EPW = 10752
EP = NW * EPW
PAD_IDX = N

CHD = 128
NCHD = EPW // CHD
CH = 128
NCH = EPW // CH
NBUF = 3
IB = 4
NACC = 10104
RPTA = 632
RPTL = NACC - 15 * RPTA


def _deg_body(dst_hbm, ones_hbm, zeros_hbm, deg_out, dst_all, ones_v, acc):
    c = lax.axis_index("c")
    s = lax.axis_index("s")
    wid = s * NC + c
    pltpu.sync_copy(zeros_hbm, acc.at[pl.ds(s * RPT, RPT)])
    pltpu.sync_copy(ones_hbm, ones_v)
    pltpu.sync_copy(dst_hbm.at[wid], dst_all)
    plsc.subcore_barrier()

    def body(i, carry):
        pltpu.sync_copy(ones_v, acc.at[dst_all.at[i]], add=True)
        return carry

    lax.fori_loop(0, NCHD, body, 0)
    plsc.subcore_barrier()
    pltpu.sync_copy(acc.at[pl.ds(s * RPT, RPT)],
                    deg_out.at[c, pl.ds(s * RPT, RPT)])


@functools.cache
def _deg_kernel():
    mesh = plsc.VectorSubcoreMesh(core_axis_name="c", subcore_axis_name="s",
                                  num_cores=NC, num_subcores=NS)
    return pl.kernel(
        _deg_body,
        out_type=jax.ShapeDtypeStruct((NC, NP, D), jnp.float32),
        mesh=mesh,
        scratch_types=[
            pltpu.VMEM((NCHD, CHD), jnp.int32),
            pltpu.VMEM((CHD, D), jnp.float32),
            pltpu.VMEM_SHARED((NP, D), jnp.float32),
        ],
    )


def _spmm_body(y_hbm, src_hbm, dst_hbm, zeros_hbm, s_out,
               src_v, dst_v, rows, isem_s, isem_d, gsem, ssem, acc):
    c = lax.axis_index("c")
    s = lax.axis_index("s")
    wid = s * NC + c
    @pl.when(s < 15)
    def _():
        pltpu.sync_copy(zeros_hbm, acc.at[pl.ds(s * RPTA, RPTA)])

    @pl.when(s == 15)
    def _():
        pltpu.sync_copy(zeros_hbm.at[pl.ds(0, RPTL)],
                        acc.at[pl.ds(15 * RPTA, RPTL)])

    plsc.subcore_barrier()

    def idx_start(i, b):
        pltpu.async_copy(src_hbm.at[c, wid, i], src_v[b], isem_s[b])
        pltpu.async_copy(dst_hbm.at[wid, i], dst_v[b], isem_d[b])

    def idx_wait(i, b):
        pltpu.make_async_copy(src_hbm.at[c, wid, i], src_v[b], isem_s[b]).wait()
        pltpu.make_async_copy(dst_hbm.at[wid, i], dst_v[b], isem_d[b]).wait()

    def gather_start(b3, b4):
        pltpu.async_copy(y_hbm.at[src_v[b4]], rows[b3], gsem[b3])

    def gather_wait(b3, b4):
        pltpu.make_async_copy(y_hbm.at[src_v[b4]], rows[b3], gsem[b3]).wait()

    def scatter_start(b3, b4):
        pltpu.async_copy(rows[b3], acc.at[dst_v[b4]], ssem[b3], add=True)

    def scatter_wait(b3, b4):
        pltpu.make_async_copy(rows[b3], acc.at[dst_v[b4]], ssem[b3]).wait()

    idx_start(0, 0)
    idx_start(1, 1)
    idx_wait(0, 0)
    gather_start(0, 0)
    idx_start(2, 2)
    idx_wait(1, 1)
    gather_start(1, 1)

    def body(j, carry):
        for k in range(12):
            i = j * 12 + k
            b3 = k % 3
            b4 = k % 4
            gather_wait(b3, b4)
            scatter_start(b3, b4)

            @pl.when(i + 2 < NCH)
            def _():
                idx_wait(i + 2, (k + 2) % 4)

            @pl.when(i >= 1)
            def _():
                scatter_wait((k + 2) % 3, (k + 3) % 4)

            @pl.when(i + 2 < NCH)
            def _():
                gather_start((k + 2) % 3, (k + 2) % 4)

            @pl.when(i + 3 < NCH)
            def _():
                idx_start(i + 3, (k + 3) % 4)

        return carry

    lax.fori_loop(0, NCH // 12, body, 0)
    scatter_wait((NCH - 1) % 3, (NCH - 1) % 4)
    plsc.subcore_barrier()

    @pl.when(s < 15)
    def _():
        pltpu.sync_copy(acc.at[pl.ds(s * RPTA, RPTA)],
                        s_out.at[c, pl.ds(s * RPTA, RPTA)])

    @pl.when(s == 15)
    def _():
        pltpu.sync_copy(acc.at[pl.ds(15 * RPTA, RPTL)],
                        s_out.at[c, pl.ds(15 * RPTA, RPTL)])


@functools.cache
def _spmm_kernel():
    mesh = plsc.VectorSubcoreMesh(core_axis_name="c", subcore_axis_name="s",
                                  num_cores=NC, num_subcores=NS)
    return pl.kernel(
        _spmm_body,
        out_type=jax.ShapeDtypeStruct((NC, NP, D), jnp.float32),
        mesh=mesh,
        scratch_types=[
            [pltpu.VMEM((CH,), jnp.int32) for _ in range(IB)],
            [pltpu.VMEM((CH,), jnp.int32) for _ in range(IB)],
            [pltpu.VMEM((CH, D), jnp.float32) for _ in range(NBUF)],
            [pltpu.SemaphoreType.DMA for _ in range(IB)],
            [pltpu.SemaphoreType.DMA for _ in range(IB)],
            [pltpu.SemaphoreType.DMA for _ in range(NBUF)],
            [pltpu.SemaphoreType.DMA for _ in range(NBUF)],
            pltpu.VMEM_SHARED((NACC, D), jnp.float32),
        ],
    )


BR = 2048
GRID = NP // BR


def _dis(deg_blk):
    return lax.rsqrt(deg_blk[0, :, 0:1] + deg_blk[1, :, 0:1] + 1.0)


def _mm1_body(deg_ref, x_ref, w_ref, y_ref, dis_ref):
    dis = _dis(deg_ref[...])
    res = dis * jnp.dot(x_ref[...], w_ref[...],
                        preferred_element_type=jnp.float32)
    y_ref[0] = res
    y_ref[1] = res
    dis_ref[...] = jnp.broadcast_to(dis, (BR, D))


_mm1 = pl.pallas_call(
    _mm1_body,
    grid=(GRID,),
    in_specs=[
        pl.BlockSpec((2, BR, D), lambda i: (0, i, 0)),
        pl.BlockSpec((BR, D), lambda i: (i, 0)),
        pl.BlockSpec((D, D), lambda i: (0, 0)),
    ],
    out_specs=[pl.BlockSpec((2, BR, D), lambda i: (0, i, 0)),
               pl.BlockSpec((BR, D), lambda i: (i, 0))],
    out_shape=[jax.ShapeDtypeStruct((NC, NP, D), jnp.float32),
               jax.ShapeDtypeStruct((NP, D), jnp.float32)],
)


def _mid_body(dis_ref, s_ref, y_ref, b_ref, w_ref, o_ref):
    dis = dis_ref[...]
    h = dis * (s_ref[0] + s_ref[1] + y_ref[0]) + b_ref[...]
    h = jnp.maximum(h, 0.0)
    res = dis * jnp.dot(h, w_ref[...], preferred_element_type=jnp.float32)
    o_ref[0] = res
    o_ref[1] = res


_mid = pl.pallas_call(
    _mid_body,
    grid=(GRID,),
    in_specs=[
        pl.BlockSpec((BR, D), lambda i: (i, 0)),
        pl.BlockSpec((2, BR, D), lambda i: (0, i, 0)),
        pl.BlockSpec((1, BR, D), lambda i: (0, i, 0)),
        pl.BlockSpec((1, D), lambda i: (0, 0)),
        pl.BlockSpec((D, D), lambda i: (0, 0)),
    ],
    out_specs=pl.BlockSpec((2, BR, D), lambda i: (0, i, 0)),
    out_shape=jax.ShapeDtypeStruct((NC, NP, D), jnp.float32),
)


def _fin_body(dis_ref, s_ref, y_ref, b_ref, o_ref):
    o_ref[...] = dis_ref[...] * (s_ref[0] + s_ref[1] + y_ref[0]) + b_ref[...]


_fin = pl.pallas_call(
    _fin_body,
    grid=(GRID,),
    in_specs=[
        pl.BlockSpec((BR, D), lambda i: (i, 0)),
        pl.BlockSpec((2, BR, D), lambda i: (0, i, 0)),
        pl.BlockSpec((1, BR, D), lambda i: (0, i, 0)),
        pl.BlockSpec((1, D), lambda i: (0, 0)),
    ],
    out_specs=pl.BlockSpec((BR, D), lambda i: (i, 0)),
    out_shape=jax.ShapeDtypeStruct((NP, D), jnp.float32),
)


def kernel(x, edge_index, W1, b1, W2, b2, W3, b3):
    src = edge_index[0]
    dst = edge_index[1]
    pad_e = PAD_IDX + jnp.arange(EP - E, dtype=jnp.int32) % (NACC - N)
    src_f = jnp.concatenate([src, pad_e]).reshape(NW, NCH, CH)
    src_p = jnp.stack([src_f, src_f + NP])
    dst_p = jnp.concatenate([dst, pad_e]).reshape(NW, NCH, CH)
    dst_w = jnp.concatenate([dst, pad_e]).reshape(NW, NCHD, CHD)
    x_p = jnp.zeros((NP, D), jnp.float32).at[:N].set(x)
    onesD = jnp.ones((CHD, D), jnp.float32)
    zerosD = jnp.zeros((RPT, D), jnp.float32)
    zerosA = jnp.zeros((RPTA, D), jnp.float32)

    deg = _deg_kernel()(dst_w, onesD, zerosD)
    spmm = _spmm_kernel()
    y1, dis = _mm1(deg, x_p, W1)
    s1 = spmm(y1.reshape(NC * NP, D), src_p, dst_p, zerosA)
    y2 = _mid(dis, s1, y1, b1.reshape(1, D), W2)
    s2 = spmm(y2.reshape(NC * NP, D), src_p, dst_p, zerosA)
    y3 = _mid(dis, s2, y2, b2.reshape(1, D), W3)
    s3 = spmm(y3.reshape(NC * NP, D), src_p, dst_p, zerosA)
    out = _fin(dis, s3, y3, b3.reshape(1, D))
    return out[:N]


# speedup vs baseline: 1.0656x; 1.0044x over previous
import functools

import jax
import jax.numpy as jnp
from jax import lax
from jax.experimental import pallas as pl
from jax.experimental.pallas import tpu as pltpu
from jax.experimental.pallas import tpu_sc as plsc

N = 10000
D = 128
E = 320000

NC = 2
NS = 16
NW = NC * NS
NP = 10240
RPT = NP // NS
EPW = 10752
EP = NW * EPW
PAD_IDX = N

CHD = 128
NCHD = EPW // CHD
CH = 128
NCH = EPW // CH
NBUF = 3
IB = 4
NACC = 10104
RPTA = 632
RPTL = NACC - 15 * RPTA


def _deg_body(dst_hbm, ones_hbm, zeros_hbm, deg_out, dst_all, ones_v, acc):
    c = lax.axis_index("c")
    s = lax.axis_index("s")
    wid = s * NC + c
    pltpu.sync_copy(zeros_hbm, acc.at[pl.ds(s * RPT, RPT)])
    pltpu.sync_copy(ones_hbm, ones_v)
    pltpu.sync_copy(dst_hbm.at[wid], dst_all)
    plsc.subcore_barrier()

    def body(i, carry):
        pltpu.sync_copy(ones_v, acc.at[dst_all.at[i]], add=True)
        return carry

    lax.fori_loop(0, NCHD, body, 0)
    plsc.subcore_barrier()
    pltpu.sync_copy(acc.at[pl.ds(s * RPT, RPT)],
                    deg_out.at[c, pl.ds(s * RPT, RPT)])


@functools.cache
def _deg_kernel():
    mesh = plsc.VectorSubcoreMesh(core_axis_name="c", subcore_axis_name="s",
                                  num_cores=NC, num_subcores=NS)
    return pl.kernel(
        _deg_body,
        out_type=jax.ShapeDtypeStruct((NC, NP, D), jnp.float32),
        mesh=mesh,
        scratch_types=[
            pltpu.VMEM((NCHD, CHD), jnp.int32),
            pltpu.VMEM((CHD, D), jnp.float32),
            pltpu.VMEM_SHARED((NP, D), jnp.float32),
        ],
    )


def _spmm_body(y_hbm, src_hbm, dst_hbm, zeros_hbm, s_out,
               src_v, dst_v, rows, isem_s, isem_d, gsem, ssem, acc):
    c = lax.axis_index("c")
    s = lax.axis_index("s")
    wid = s * NC + c
    @pl.when(s < 15)
    def _():
        pltpu.sync_copy(zeros_hbm, acc.at[pl.ds(s * RPTA, RPTA)])

    @pl.when(s == 15)
    def _():
        pltpu.sync_copy(zeros_hbm.at[pl.ds(0, RPTL)],
                        acc.at[pl.ds(15 * RPTA, RPTL)])

    plsc.subcore_barrier()

    def idx_start(i, b):
        pltpu.async_copy(src_hbm.at[c, wid, i], src_v[b], isem_s[b])
        pltpu.async_copy(dst_hbm.at[wid, i], dst_v[b], isem_d[b])

    def idx_wait(i, b):
        pltpu.make_async_copy(src_hbm.at[c, wid, i], src_v[b], isem_s[b]).wait()
        pltpu.make_async_copy(dst_hbm.at[wid, i], dst_v[b], isem_d[b]).wait()

    def gather_start(b3, b4):
        pltpu.async_copy(y_hbm.at[src_v[b4]], rows[b3], gsem[b3])

    def gather_wait(b3, b4):
        pltpu.make_async_copy(y_hbm.at[src_v[b4]], rows[b3], gsem[b3]).wait()

    def scatter_start(b3, b4):
        pltpu.async_copy(rows[b3], acc.at[dst_v[b4]], ssem[b3], add=True)

    def scatter_wait(b3, b4):
        pltpu.make_async_copy(rows[b3], acc.at[dst_v[b4]], ssem[b3]).wait()

    idx_start(0, 0)
    idx_start(1, 1)
    idx_wait(0, 0)
    gather_start(0, 0)
    idx_start(2, 2)
    idx_wait(1, 1)
    gather_start(1, 1)

    def body(j, carry):
        for k in range(12):
            i = j * 12 + k
            b3 = k % 3
            b4 = k % 4
            gather_wait(b3, b4)
            scatter_start(b3, b4)

            @pl.when(i + 2 < NCH)
            def _():
                idx_wait(i + 2, (k + 2) % 4)

            @pl.when(i >= 1)
            def _():
                scatter_wait((k + 2) % 3, (k + 3) % 4)

            @pl.when(i + 2 < NCH)
            def _():
                gather_start((k + 2) % 3, (k + 2) % 4)

            @pl.when(i + 3 < NCH)
            def _():
                idx_start(i + 3, (k + 3) % 4)

        return carry

    lax.fori_loop(0, NCH // 12, body, 0)
    scatter_wait((NCH - 1) % 3, (NCH - 1) % 4)
    plsc.subcore_barrier()

    @pl.when(s < 15)
    def _():
        pltpu.sync_copy(acc.at[pl.ds(s * RPTA, RPTA)],
                        s_out.at[c, pl.ds(s * RPTA, RPTA)])

    @pl.when(s == 15)
    def _():
        pltpu.sync_copy(acc.at[pl.ds(15 * RPTA, RPTL)],
                        s_out.at[c, pl.ds(15 * RPTA, RPTL)])


@functools.cache
def _spmm_kernel():
    mesh = plsc.VectorSubcoreMesh(core_axis_name="c", subcore_axis_name="s",
                                  num_cores=NC, num_subcores=NS)
    return pl.kernel(
        _spmm_body,
        out_type=jax.ShapeDtypeStruct((NC, NP, D), jnp.float32),
        mesh=mesh,
        scratch_types=[
            [pltpu.VMEM((CH,), jnp.int32) for _ in range(IB)],
            [pltpu.VMEM((CH,), jnp.int32) for _ in range(IB)],
            [pltpu.VMEM((CH, D), jnp.float32) for _ in range(NBUF)],
            [pltpu.SemaphoreType.DMA for _ in range(IB)],
            [pltpu.SemaphoreType.DMA for _ in range(IB)],
            [pltpu.SemaphoreType.DMA for _ in range(NBUF)],
            [pltpu.SemaphoreType.DMA for _ in range(NBUF)],
            pltpu.VMEM_SHARED((NACC, D), jnp.float32),
        ],
    )


BR = 2560
GRID = NP // BR


def _dis(deg_blk):
    return lax.rsqrt(deg_blk[0, :, 0:1] + deg_blk[1, :, 0:1] + 1.0)


def _mm1_body(deg_ref, x_ref, w_ref, y_ref, dis_ref):
    dis = _dis(deg_ref[...])
    res = dis * jnp.dot(x_ref[...], w_ref[...],
                        preferred_element_type=jnp.float32)
    y_ref[0] = res
    y_ref[1] = res
    dis_ref[...] = jnp.broadcast_to(dis, (BR, D))


_mm1 = pl.pallas_call(
    _mm1_body,
    grid=(GRID,),
    in_specs=[
        pl.BlockSpec((2, BR, D), lambda i: (0, i, 0)),
        pl.BlockSpec((BR, D), lambda i: (i, 0)),
        pl.BlockSpec((D, D), lambda i: (0, 0)),
    ],
    out_specs=[pl.BlockSpec((2, BR, D), lambda i: (0, i, 0)),
               pl.BlockSpec((BR, D), lambda i: (i, 0))],
    out_shape=[jax.ShapeDtypeStruct((NC, NP, D), jnp.float32),
               jax.ShapeDtypeStruct((NP, D), jnp.float32)],
)


def _mid_body(dis_ref, s_ref, y_ref, b_ref, w_ref, o_ref):
    dis = dis_ref[...]
    h = dis * (s_ref[0] + s_ref[1] + y_ref[0]) + b_ref[...]
    h = jnp.maximum(h, 0.0)
    res = dis * jnp.dot(h, w_ref[...], preferred_element_type=jnp.float32)
    o_ref[0] = res
    o_ref[1] = res


_mid = pl.pallas_call(
    _mid_body,
    grid=(GRID,),
    in_specs=[
        pl.BlockSpec((BR, D), lambda i: (i, 0)),
        pl.BlockSpec((2, BR, D), lambda i: (0, i, 0)),
        pl.BlockSpec((1, BR, D), lambda i: (0, i, 0)),
        pl.BlockSpec((1, D), lambda i: (0, 0)),
        pl.BlockSpec((D, D), lambda i: (0, 0)),
    ],
    out_specs=pl.BlockSpec((2, BR, D), lambda i: (0, i, 0)),
    out_shape=jax.ShapeDtypeStruct((NC, NP, D), jnp.float32),
)


def _fin_body(dis_ref, s_ref, y_ref, b_ref, o_ref):
    o_ref[...] = dis_ref[...] * (s_ref[0] + s_ref[1] + y_ref[0]) + b_ref[...]


_fin = pl.pallas_call(
    _fin_body,
    grid=(GRID,),
    in_specs=[
        pl.BlockSpec((BR, D), lambda i: (i, 0)),
        pl.BlockSpec((2, BR, D), lambda i: (0, i, 0)),
        pl.BlockSpec((1, BR, D), lambda i: (0, i, 0)),
        pl.BlockSpec((1, D), lambda i: (0, 0)),
    ],
    out_specs=pl.BlockSpec((BR, D), lambda i: (i, 0)),
    out_shape=jax.ShapeDtypeStruct((NP, D), jnp.float32),
)


def kernel(x, edge_index, W1, b1, W2, b2, W3, b3):
    src = edge_index[0]
    dst = edge_index[1]
    pad_e = PAD_IDX + jnp.arange(EP - E, dtype=jnp.int32) % (NACC - N)
    src_f = jnp.concatenate([src, pad_e]).reshape(NW, NCH, CH)
    src_p = jnp.stack([src_f, src_f + NP])
    dst_p = jnp.concatenate([dst, pad_e]).reshape(NW, NCH, CH)
    dst_w = jnp.concatenate([dst, pad_e]).reshape(NW, NCHD, CHD)
    x_p = jnp.zeros((NP, D), jnp.float32).at[:N].set(x)
    onesD = jnp.ones((CHD, D), jnp.float32)
    zerosD = jnp.zeros((RPT, D), jnp.float32)
    zerosA = jnp.zeros((RPTA, D), jnp.float32)

    deg = _deg_kernel()(dst_w, onesD, zerosD)
    spmm = _spmm_kernel()
    y1, dis = _mm1(deg, x_p, W1)
    s1 = spmm(y1.reshape(NC * NP, D), src_p, dst_p, zerosA)
    y2 = _mid(dis, s1, y1, b1.reshape(1, D), W2)
    s2 = spmm(y2.reshape(NC * NP, D), src_p, dst_p, zerosA)
    y3 = _mid(dis, s2, y2, b2.reshape(1, D), W3)
    s3 = spmm(y3.reshape(NC * NP, D), src_p, dst_p, zerosA)
    out = _fin(dis, s3, y3, b3.reshape(1, D))
    return out[:N]
